# baseline (speedup 1.0000x reference)
import functools

import jax
import jax.numpy as jnp
from jax.experimental import pallas as pl
from jax.experimental.pallas import tpu as pltpu

_TB = 64
_UNROLL = 16


def _lstm_gates(g, H):
    i = jax.nn.sigmoid(g[:, 0:H])
    f = jax.nn.sigmoid(g[:, H:2 * H])
    gg = jnp.tanh(g[:, 2 * H:3 * H])
    o = jax.nn.sigmoid(g[:, 3 * H:4 * H])
    return i, f, gg, o


def _lstm_body(len_ref, x_ref, wx0_ref, wh0_ref, b0_ref, wx1_ref, wh1_ref,
               b1_ref, ctx_ref, gx0_ref, gx1_ref, o0b_ref,
               h0_ref, c0_ref, h1_ref, c1_ref):
    blk = pl.program_id(0)
    nblk = pl.num_programs(0) - 1
    B = len_ref.shape[0]
    H = wh0_ref.shape[0]
    bf16 = jnp.bfloat16

    @pl.when(blk == 0)
    def _():
        h0_ref[...] = jnp.zeros_like(h0_ref)
        c0_ref[...] = jnp.zeros_like(c0_ref)
        h1_ref[...] = jnp.zeros_like(h1_ref)
        c1_ref[...] = jnp.zeros_like(c1_ref)
        o0b_ref[...] = jnp.zeros_like(o0b_ref)

    lens = len_ref[...]
    ml = jnp.max(lens).astype(jnp.int32)
    lo0 = blk * _TB
    lo1 = (blk - 1) * _TB

    cnt0 = jnp.clip(ml - lo0, 0, _TB)

    @pl.when(jnp.logical_and(blk < nblk, cnt0 > 0))
    def _():
        x = x_ref[...].reshape(_TB * B, x_ref.shape[2])
        gx0_ref[...] = (
            jnp.dot(x, wx0_ref[...], preferred_element_type=jnp.float32)
            + b0_ref[...]
        ).astype(gx0_ref.dtype)

    prev = o0b_ref[pl.ds(((blk - 1) % 2) * _TB * B, _TB * B), :]
    gx1_ref[...] = (
        jnp.dot(prev, wx1_ref[...], preferred_element_type=jnp.float32)
        + b1_ref[...]
    ).astype(gx1_ref.dtype)

    ctx_ref[pl.ds(jnp.maximum(lo1, 0), _TB), :, :] = jnp.zeros(
        (_TB, B, ctx_ref.shape[2]), ctx_ref.dtype)

    f0_ = lo0.astype(jnp.float32)
    f1_ = lo1.astype(jnp.float32)
    obase = (blk % 2) * _TB * B

    def step(t, carry):
        h0, c0, h1, c1 = carry
        tf = t.astype(jnp.float32)

        m0 = (f0_ + tf < lens).astype(jnp.float32)
        g0 = gx0_ref[pl.ds(t * B, B), :].astype(jnp.float32) + jnp.dot(
            h0.astype(bf16), wh0_ref[...], preferred_element_type=jnp.float32)
        i0, f0, gg0, o0 = _lstm_gates(g0, H)
        c0n = f0 * c0 + i0 * gg0
        h0n = o0 * jnp.tanh(c0n)
        o0b_ref[pl.ds(obase + t * B, B), :] = (m0 * h0n).astype(bf16)
        c0 = m0 * c0n + (1.0 - m0) * c0
        h0 = m0 * h0n + (1.0 - m0) * h0

        t1 = f1_ + tf
        m1 = jnp.logical_and(t1 >= 0.0, t1 < lens).astype(jnp.float32)
        g1 = gx1_ref[pl.ds(t * B, B), :].astype(jnp.float32) + jnp.dot(
            h1.astype(bf16), wh1_ref[...], preferred_element_type=jnp.float32)
        i1, f1, gg1, o1 = _lstm_gates(g1, H)
        c1n = f1 * c1 + i1 * gg1
        h1n = o1 * jnp.tanh(c1n)
        ctx_ref[jnp.maximum(lo1 + t, 0), :, :] = m1 * h1n
        c1 = m1 * c1n + (1.0 - m1) * c1
        h1 = m1 * h1n + (1.0 - m1) * h1
        return h0, c0, h1, c1

    def stepn(u, carry):
        for j in range(_UNROLL):
            carry = step(_UNROLL * u + j, carry)
        return carry

    cnt = jnp.clip(ml - jnp.maximum(blk - 1, 0) * _TB, 0, _TB)
    nch = (cnt + _UNROLL - 1) // _UNROLL
    carry = (h0_ref[...], c0_ref[...], h1_ref[...], c1_ref[...])
    h0, c0, h1, c1 = jax.lax.fori_loop(0, nch, stepn, carry)
    h0_ref[...] = h0
    c0_ref[...] = c0
    h1_ref[...] = h1
    c1_ref[...] = c1


def _fusion_body(code_ref, ctx_ref, u_ref, v_ref, hm_ref, fl_ref, w_ref):
    cb = code_ref[0]
    xb = ctx_ref[...]
    S = cb.shape[0]
    OUT = u_ref.shape[1]
    K = hm_ref.shape[0]

    v = jnp.tanh(jnp.dot(cb, u_ref[...], preferred_element_type=jnp.float32))
    q = jnp.tanh(jnp.dot(xb, v_ref[...], preferred_element_type=jnp.float32))

    qb = q.astype(jnp.bfloat16)
    fl = jnp.zeros((1, OUT), jnp.float32)
    wk = jnp.zeros((S, 1), jnp.float32)
    for k in range(K):
        hk = hm_ref[k:k + 1, :]
        vk = v * hk
        att = jax.lax.dot_general(
            vk.astype(jnp.bfloat16), qb, (((1,), (1,)), ((), ())),
            preferred_element_type=jnp.float32)
        mx = jnp.max(att, axis=1, keepdims=True)
        e = jnp.exp(att - mx)
        z = jnp.sum(e, axis=1, keepdims=True)
        p = e / z
        diag = jnp.sum(vk * q, axis=1, keepdims=True)
        wk = wk + jnp.exp(diag - mx) / z
        t_mat = jnp.dot(p.astype(jnp.bfloat16), qb,
                        preferred_element_type=jnp.float32)
        fl = fl + jnp.sum(v * t_mat, axis=0, keepdims=True)
    w = wk / jnp.sum(wk)
    fl_ref[...] = fl.reshape(1, 1, OUT)
    w_ref[...] = w.reshape(1, 1, S)


@functools.partial(jax.jit, static_argnames=("interpret",))
def _run(code_tensor, lengths, W_ih0, W_hh0, b_ih0, b_hh0, W_ih1, W_hh1,
         b_ih1, b_hh1, U, V, h_mat, interpret=False):
    B, S, D = code_tensor.shape
    H = W_hh0.shape[1]
    OUT = U.shape[1]
    K = h_mat.shape[0]
    f32 = jnp.float32
    bf16 = jnp.bfloat16

    lens = lengths.astype(f32).reshape(B, 1)
    x_t = jnp.transpose(code_tensor, (1, 0, 2))
    b0 = (b_ih0 + b_hh0).reshape(1, 4 * H)
    b1 = (b_ih1 + b_hh1).reshape(1, 4 * H)
    wx0 = W_ih0.T
    wh0 = W_hh0.T.astype(bf16)
    wx1 = W_ih1.T.astype(bf16)
    wh1 = W_hh1.T.astype(bf16)

    nblk = S // _TB
    last = nblk - 1
    ctx_t = pl.pallas_call(
        _lstm_body,
        grid=(nblk + 1,),
        in_specs=[
            pl.BlockSpec((B, 1), lambda i: (0, 0)),
            pl.BlockSpec((_TB, B, D), lambda i: (jnp.minimum(i, last), 0, 0)),
            pl.BlockSpec(wx0.shape, lambda i: (0, 0)),
            pl.BlockSpec(wh0.shape, lambda i: (0, 0)),
            pl.BlockSpec(b0.shape, lambda i: (0, 0)),
            pl.BlockSpec(wx1.shape, lambda i: (0, 0)),
            pl.BlockSpec(wh1.shape, lambda i: (0, 0)),
            pl.BlockSpec(b1.shape, lambda i: (0, 0)),
        ],
        out_specs=pl.BlockSpec((S, B, H), lambda i: (0, 0, 0)),
        out_shape=jax.ShapeDtypeStruct((S, B, H), f32),
        scratch_shapes=[
            pltpu.VMEM((_TB * B, 4 * H), bf16),
            pltpu.VMEM((_TB * B, 4 * H), bf16),
            pltpu.VMEM((2 * _TB * B, H), bf16),
            pltpu.VMEM((B, H), f32),
            pltpu.VMEM((B, H), f32),
            pltpu.VMEM((B, H), f32),
            pltpu.VMEM((B, H), f32),
        ],
        interpret=interpret,
    )(lens, x_t, wx0, wh0, b0, wx1, wh1, b1)

    ctx2 = ctx_t.reshape(S, B * H)

    file_level, w = pl.pallas_call(
        _fusion_body,
        grid=(B,),
        in_specs=[
            pl.BlockSpec((1, S, D), lambda b: (b, 0, 0)),
            pl.BlockSpec((S, H), lambda b: (0, b)),
            pl.BlockSpec(U.shape, lambda b: (0, 0)),
            pl.BlockSpec(V.shape, lambda b: (0, 0)),
            pl.BlockSpec(h_mat.shape, lambda b: (0, 0)),
        ],
        out_specs=[
            pl.BlockSpec((1, 1, OUT), lambda b: (b, 0, 0)),
            pl.BlockSpec((1, 1, S), lambda b: (b, 0, 0)),
        ],
        out_shape=[
            jax.ShapeDtypeStruct((B, 1, OUT), f32),
            jax.ShapeDtypeStruct((B, 1, S), f32),
        ],
        interpret=interpret,
    )(code_tensor, ctx2, U, V, h_mat)

    return file_level.reshape(B, OUT), w.reshape(B, S)


def kernel(code_tensor, lengths, W_ih0, W_hh0, b_ih0, b_hh0, W_ih1, W_hh1,
           b_ih1, b_hh1, U, V, h_mat):
    return _run(code_tensor, lengths, W_ih0, W_hh0, b_ih0, b_hh0,
                W_ih1, W_hh1, b_ih1, b_hh1, U, V, h_mat)


# speedup vs baseline: 1.0111x; 1.0111x over previous
import functools

import jax
import jax.numpy as jnp
from jax.experimental import pallas as pl
from jax.experimental.pallas import tpu as pltpu

_TB = 64
_UNROLL = 16


def _lstm_gates(g, H):
    i = jax.nn.sigmoid(g[:, 0:H])
    f = jax.nn.sigmoid(g[:, H:2 * H])
    gg = jnp.tanh(g[:, 2 * H:3 * H])
    o = jax.nn.sigmoid(g[:, 3 * H:4 * H])
    return i, f, gg, o


def _lstm_body(len_ref, x_ref, wx0_ref, wh0_ref, b0_ref, wx1_ref, wh1_ref,
               b1_ref, ctx_ref, gx0_ref, gx1_ref, o0b_ref,
               h0_ref, c0_ref, h1_ref, c1_ref):
    blk = pl.program_id(0)
    nblk = pl.num_programs(0) - 1
    B = len_ref.shape[0]
    H = wh0_ref.shape[0]
    bf16 = jnp.bfloat16

    @pl.when(blk == 0)
    def _():
        h0_ref[...] = jnp.zeros_like(h0_ref)
        c0_ref[...] = jnp.zeros_like(c0_ref)
        h1_ref[...] = jnp.zeros_like(h1_ref)
        c1_ref[...] = jnp.zeros_like(c1_ref)
        o0b_ref[...] = jnp.zeros_like(o0b_ref)

    lens = len_ref[...]
    ml = jnp.max(lens).astype(jnp.int32)
    lo0 = blk * _TB
    lo1 = (blk - 1) * _TB

    cnt0 = jnp.clip(ml - lo0, 0, _TB)

    @pl.when(jnp.logical_and(blk < nblk, cnt0 > 0))
    def _():
        x = x_ref[...].reshape(_TB * B, x_ref.shape[2])
        gx0_ref[...] = (
            jnp.dot(x, wx0_ref[...], preferred_element_type=jnp.float32)
            + b0_ref[...]
        ).astype(gx0_ref.dtype)

    prev = o0b_ref[pl.ds(((blk - 1) % 2) * _TB * B, _TB * B), :]
    gx1_ref[...] = (
        jnp.dot(prev, wx1_ref[...], preferred_element_type=jnp.float32)
        + b1_ref[...]
    ).astype(gx1_ref.dtype)

    ctx_ref[pl.ds(jnp.maximum(lo1, 0), _TB), :, :] = jnp.zeros(
        (_TB, B, ctx_ref.shape[2]), ctx_ref.dtype)

    f0_ = lo0.astype(jnp.float32)
    f1_ = lo1.astype(jnp.float32)
    obase = (blk % 2) * _TB * B

    def step(t, carry):
        h0, c0, h1, c1 = carry
        tf = t.astype(jnp.float32)

        m0 = (f0_ + tf < lens).astype(jnp.float32)
        g0 = gx0_ref[pl.ds(t * B, B), :].astype(jnp.float32) + jnp.dot(
            h0.astype(bf16), wh0_ref[...], preferred_element_type=jnp.float32)
        i0, f0, gg0, o0 = _lstm_gates(g0, H)
        c0n = f0 * c0 + i0 * gg0
        h0n = o0 * jnp.tanh(c0n)
        o0b_ref[pl.ds(obase + t * B, B), :] = (m0 * h0n).astype(bf16)
        c0 = m0 * c0n + (1.0 - m0) * c0
        h0 = m0 * h0n + (1.0 - m0) * h0

        t1 = f1_ + tf
        m1 = jnp.logical_and(t1 >= 0.0, t1 < lens).astype(jnp.float32)
        g1 = gx1_ref[pl.ds(t * B, B), :].astype(jnp.float32) + jnp.dot(
            h1.astype(bf16), wh1_ref[...], preferred_element_type=jnp.float32)
        i1, f1, gg1, o1 = _lstm_gates(g1, H)
        c1n = f1 * c1 + i1 * gg1
        h1n = o1 * jnp.tanh(c1n)
        ctx_ref[jnp.maximum(lo1 + t, 0), :, :] = m1 * h1n
        c1 = m1 * c1n + (1.0 - m1) * c1
        h1 = m1 * h1n + (1.0 - m1) * h1
        return h0, c0, h1, c1

    def stepn(u, carry):
        for j in range(_UNROLL):
            carry = step(_UNROLL * u + j, carry)
        return carry

    cnt = jnp.clip(ml - jnp.maximum(blk - 1, 0) * _TB, 0, _TB)
    nch = (cnt + _UNROLL - 1) // _UNROLL
    carry = (h0_ref[...], c0_ref[...], h1_ref[...], c1_ref[...])
    h0, c0, h1, c1 = jax.lax.fori_loop(0, nch, stepn, carry)
    h0_ref[...] = h0
    c0_ref[...] = c0
    h1_ref[...] = h1
    c1_ref[...] = c1


def _fusion_body(code_ref, ctx_ref, u_ref, v_ref, hm_ref, fl_ref, w_ref):
    cb = code_ref[0]
    xb = ctx_ref[...]
    S = cb.shape[0]
    OUT = u_ref.shape[1]
    K = hm_ref.shape[0]

    v = jnp.tanh(jnp.dot(cb, u_ref[...], preferred_element_type=jnp.float32))
    q = jnp.tanh(jnp.dot(xb, v_ref[...], preferred_element_type=jnp.float32))

    qb = q.astype(jnp.bfloat16)
    fl = jnp.zeros((1, OUT), jnp.float32)
    wk = jnp.zeros((S, 1), jnp.float32)
    for k in range(K):
        hk = hm_ref[k:k + 1, :]
        vk = v * hk
        att = jax.lax.dot_general(
            vk.astype(jnp.bfloat16), qb, (((1,), (1,)), ((), ())),
            preferred_element_type=jnp.float32)
        mx = jnp.max(att, axis=1, keepdims=True)
        e = jnp.exp(att - mx)
        z = jnp.sum(e, axis=1, keepdims=True)
        p = e / z
        diag = jnp.sum(vk * q, axis=1, keepdims=True)
        wk = wk + jnp.exp(diag - mx) / z
        t_mat = jnp.dot(p, q, preferred_element_type=jnp.float32)
        fl = fl + jnp.sum(v * t_mat, axis=0, keepdims=True)
    w = wk / jnp.sum(wk)
    fl_ref[...] = fl.reshape(1, 1, OUT)
    w_ref[...] = w.reshape(1, 1, S)


@functools.partial(jax.jit, static_argnames=("interpret",))
def _run(code_tensor, lengths, W_ih0, W_hh0, b_ih0, b_hh0, W_ih1, W_hh1,
         b_ih1, b_hh1, U, V, h_mat, interpret=False):
    B, S, D = code_tensor.shape
    H = W_hh0.shape[1]
    OUT = U.shape[1]
    K = h_mat.shape[0]
    f32 = jnp.float32
    bf16 = jnp.bfloat16

    lens = lengths.astype(f32).reshape(B, 1)
    x_t = jnp.transpose(code_tensor, (1, 0, 2))
    b0 = (b_ih0 + b_hh0).reshape(1, 4 * H)
    b1 = (b_ih1 + b_hh1).reshape(1, 4 * H)
    wx0 = W_ih0.T
    wh0 = W_hh0.T.astype(bf16)
    wx1 = W_ih1.T.astype(bf16)
    wh1 = W_hh1.T.astype(bf16)

    nblk = S // _TB
    last = nblk - 1
    ctx_t = pl.pallas_call(
        _lstm_body,
        grid=(nblk + 1,),
        in_specs=[
            pl.BlockSpec((B, 1), lambda i: (0, 0)),
            pl.BlockSpec((_TB, B, D), lambda i: (jnp.minimum(i, last), 0, 0)),
            pl.BlockSpec(wx0.shape, lambda i: (0, 0)),
            pl.BlockSpec(wh0.shape, lambda i: (0, 0)),
            pl.BlockSpec(b0.shape, lambda i: (0, 0)),
            pl.BlockSpec(wx1.shape, lambda i: (0, 0)),
            pl.BlockSpec(wh1.shape, lambda i: (0, 0)),
            pl.BlockSpec(b1.shape, lambda i: (0, 0)),
        ],
        out_specs=pl.BlockSpec((S, B, H), lambda i: (0, 0, 0)),
        out_shape=jax.ShapeDtypeStruct((S, B, H), f32),
        scratch_shapes=[
            pltpu.VMEM((_TB * B, 4 * H), bf16),
            pltpu.VMEM((_TB * B, 4 * H), bf16),
            pltpu.VMEM((2 * _TB * B, H), bf16),
            pltpu.VMEM((B, H), f32),
            pltpu.VMEM((B, H), f32),
            pltpu.VMEM((B, H), f32),
            pltpu.VMEM((B, H), f32),
        ],
        interpret=interpret,
    )(lens, x_t, wx0, wh0, b0, wx1, wh1, b1)

    ctx2 = ctx_t.reshape(S, B * H)

    file_level, w = pl.pallas_call(
        _fusion_body,
        grid=(B,),
        in_specs=[
            pl.BlockSpec((1, S, D), lambda b: (b, 0, 0)),
            pl.BlockSpec((S, H), lambda b: (0, b)),
            pl.BlockSpec(U.shape, lambda b: (0, 0)),
            pl.BlockSpec(V.shape, lambda b: (0, 0)),
            pl.BlockSpec(h_mat.shape, lambda b: (0, 0)),
        ],
        out_specs=[
            pl.BlockSpec((1, 1, OUT), lambda b: (b, 0, 0)),
            pl.BlockSpec((1, 1, S), lambda b: (b, 0, 0)),
        ],
        out_shape=[
            jax.ShapeDtypeStruct((B, 1, OUT), f32),
            jax.ShapeDtypeStruct((B, 1, S), f32),
        ],
        interpret=interpret,
    )(code_tensor, ctx2, U, V, h_mat)

    return file_level.reshape(B, OUT), w.reshape(B, S)


def kernel(code_tensor, lengths, W_ih0, W_hh0, b_ih0, b_hh0, W_ih1, W_hh1,
           b_ih1, b_hh1, U, V, h_mat):
    return _run(code_tensor, lengths, W_ih0, W_hh0, b_ih0, b_hh0,
                W_ih1, W_hh1, b_ih1, b_hh1, U, V, h_mat)
